# baseline (speedup 1.0000x reference)
import functools

import jax
import jax.numpy as jnp
from jax import lax
from jax.experimental import pallas as pl
from jax.experimental.pallas import tpu as pltpu
from jax.experimental.pallas import tpu_sc as plsc

_NC = 2
_NS = 16
_NW = _NC * _NS

_CH = 128
_NB = 2


def _edge_agg_kernel(n_pad, n0, n1, d):
  slab = n_pad // _NS
  mesh = plsc.VectorSubcoreMesh(core_axis_name="c", subcore_axis_name="s")

  nbi = 6
  nbd = 8

  @functools.partial(
      pl.kernel,
      out_type=jax.ShapeDtypeStruct((_NC, n_pad, d), jnp.float32),
      mesh=mesh,
      scratch_types=[
          pltpu.VMEM((nbi, _CH), jnp.int32),
          pltpu.VMEM((nbd, _CH), jnp.int32),
          pltpu.VMEM((_NB, _CH, d), jnp.float32),
          pltpu.VMEM_SHARED((n_pad, d), jnp.float32),
          pltpu.SemaphoreType.DMA((_NB,)),
          pltpu.SemaphoreType.DMA((_NB,)),
          pltpu.SemaphoreType.DMA((nbi,)),
          pltpu.SemaphoreType.DMA((nbd,)),
      ],
  )
  def k(hs_hbm, src_hbm, dst_hbm, zeros_hbm, out_hbm, idx_s, idx_d, rows,
        acc, sem_g, sem_s, sem_i, sem_j):
    c = lax.axis_index("c")
    s = lax.axis_index("s")
    wid = c * _NS + s
    nb = jnp.where(c == 0, n0, n1)
    pltpu.sync_copy(zeros_hbm, acc.at[pl.ds(s * slab, slab)])
    for bi in range(nbi):
      pltpu.async_copy(src_hbm.at[wid, bi], idx_s.at[bi], sem_i.at[bi])
    for bj in range(nbd):
      pltpu.async_copy(dst_hbm.at[wid, bj], idx_d.at[bj], sem_j.at[bj])
    plsc.subcore_barrier()

    for b in range(_NB - 1):
      pltpu.make_async_copy(src_hbm.at[wid, b], idx_s.at[b],
                            sem_i.at[b]).wait()
      pltpu.async_copy(hs_hbm.at[idx_s.at[b]], rows.at[b], sem_g.at[b])

    @pl.loop(0, nb)
    def _chunk(j):
      b = lax.rem(j, _NB)
      bi = lax.rem(j, nbi)
      bj = lax.rem(j, nbd)
      pltpu.make_async_copy(hs_hbm.at[idx_s.at[bi]], rows.at[b],
                            sem_g.at[b]).wait()
      prv = j - 1

      @pl.when(prv >= 0)
      def _wprev():
        bp = lax.rem(prv, _NB)
        bjp = lax.rem(prv, nbd)
        pltpu.make_async_copy(rows.at[bp], acc.at[idx_d.at[bjp]],
                              sem_s.at[bp]).wait()
        nxj = prv + nbd

        @pl.when(nxj < nb)
        def _pref_di():
          pltpu.async_copy(dst_hbm.at[wid, nxj], idx_d.at[bjp],
                           sem_j.at[bjp])

      pltpu.make_async_copy(dst_hbm.at[wid, j], idx_d.at[bj],
                            sem_j.at[bj]).wait()
      pltpu.async_copy(rows.at[b], acc.at[idx_d.at[bj]], sem_s.at[b],
                       add=True)
      nxi = j + nbi

      @pl.when(nxi < nb)
      def _pref_si():
        pltpu.async_copy(src_hbm.at[wid, nxi], idx_s.at[bi], sem_i.at[bi])

      nxg = j + _NB - 1

      @pl.when(nxg < nb)
      def _pref_rows():
        bg = lax.rem(nxg, _NB)
        big = lax.rem(nxg, nbi)
        pltpu.make_async_copy(src_hbm.at[wid, nxg], idx_s.at[big],
                              sem_i.at[big]).wait()
        pltpu.async_copy(hs_hbm.at[idx_s.at[big]], rows.at[bg],
                         sem_g.at[bg])

    lst = nb - 1
    pltpu.make_async_copy(rows.at[lax.rem(lst, _NB)],
                          acc.at[idx_d.at[lax.rem(lst, nbd)]],
                          sem_s.at[lax.rem(lst, _NB)]).wait()
    plsc.subcore_barrier()
    pltpu.sync_copy(acc.at[pl.ds(s * slab, slab)],
                    out_hbm.at[c, pl.ds(s * slab, slab)])

  return k


def _degree_kernel(n_pad, n0, n1):
  n_max = max(n0, n1)
  slab = n_pad // _NS
  nbd = 8
  mesh = plsc.VectorSubcoreMesh(core_axis_name="c", subcore_axis_name="s")

  @functools.partial(
      pl.kernel,
      out_type=jax.ShapeDtypeStruct((_NC, n_pad), jnp.float32),
      mesh=mesh,
      scratch_types=[
          pltpu.VMEM((n_max, _CH), jnp.int32),
          pltpu.VMEM((_CH,), jnp.float32),
          pltpu.VMEM_SHARED((n_pad,), jnp.float32),
          pltpu.SemaphoreType.DMA((nbd,)),
      ],
  )
  def k(dst_hbm, ones_hbm, zeros_hbm, out_hbm, idx_d, ones_v, acc, sem_s):
    c = lax.axis_index("c")
    s = lax.axis_index("s")
    wid = c * _NS + s
    nb = jnp.where(c == 0, n0, n1)
    pltpu.sync_copy(dst_hbm.at[wid], idx_d)
    pltpu.sync_copy(ones_hbm, ones_v)
    pltpu.sync_copy(zeros_hbm.at[pl.ds(0, slab)],
                    acc.at[pl.ds(s * slab, slab)])
    plsc.subcore_barrier()

    for b in range(nbd):
      pltpu.async_copy(ones_v, acc.at[idx_d.at[b]], sem_s.at[b], add=True)

    @pl.loop(0, nb)
    def _chunk(j):
      b = lax.rem(j, nbd)
      pltpu.make_async_copy(ones_v, acc.at[idx_d.at[j]], sem_s.at[b]).wait()
      nxt = j + nbd

      @pl.when(nxt < nb)
      def _nx():
        pltpu.async_copy(ones_v, acc.at[idx_d.at[nxt]], sem_s.at[b],
                         add=True)

    plsc.subcore_barrier()
    pltpu.sync_copy(acc.at[pl.ds(s * slab, slab)],
                    out_hbm.at[c, pl.ds(s * slab, slab)])

  return k


def _tc_first(x_ref, w_ref, degp_ref, out_ref):
  n = x_ref.shape[0]
  deg = degp_ref[0, :n, 0:1] + degp_ref[1, :n, 0:1] + 1.0
  g = lax.rsqrt(deg)
  h = jnp.dot(x_ref[...], w_ref[...], preferred_element_type=jnp.float32)
  out_ref[...] = h * g


def _tc_mid(aggp_ref, hs_ref, degp_ref, b_ref, w_ref, out_ref, *, n):
  deg = degp_ref[0, :n, 0:1] + degp_ref[1, :n, 0:1] + 1.0
  g = lax.rsqrt(deg)
  agg = aggp_ref[0, :n, :] + aggp_ref[1, :n, :] + hs_ref[...]
  h = jnp.maximum(agg * g + b_ref[...], 0.0)
  out_ref[...] = jnp.dot(h, w_ref[...],
                         preferred_element_type=jnp.float32) * g


def _tc_last(aggp_ref, hs_ref, degp_ref, b_ref, batch_ref, out_ref,
             *, num_graphs, n):
  deg = degp_ref[0, :n, 0:1] + degp_ref[1, :n, 0:1] + 1.0
  g = lax.rsqrt(deg)
  nodes = (aggp_ref[0, :n, :] + aggp_ref[1, :n, :] + hs_ref[...]) * g \
      + b_ref[...]
  gid = lax.broadcasted_iota(jnp.int32, (num_graphs, n), 0)
  mask = (gid == batch_ref[...]).astype(jnp.float32)
  sums = jnp.dot(mask, nodes, preferred_element_type=jnp.float32)
  cnt = jnp.sum(mask, axis=1, keepdims=True)
  out_ref[...] = sums / jnp.maximum(cnt, 1.0)


def kernel(x, edge_index, batch, W1, b1, W2, b2):
  n, d_in = x.shape
  d_hid = W1.shape[1]
  d_out = W2.shape[1]
  e = edge_index.shape[1]
  num_graphs = 64

  n_pad = ((n + 1023) // 1024) * 1024
  slab = n_pad // _NS

  grain = _NW * _CH
  e_pad = ((e + grain - 1) // grain) * grain
  tot_ch = e_pad // (_NS * _CH)
  n0 = (tot_ch * 4) // 5
  n1 = tot_ch - n0
  n_mx = max(n0, n1)

  def _chunked(flat, fill):
    flat = jnp.concatenate(
        [flat, jnp.full((e_pad - e,), fill, jnp.int32)])
    a = flat[: _NS * n0 * _CH].reshape(_NS, n0, _CH)
    b = flat[_NS * n0 * _CH:].reshape(_NS, n1, _CH)
    a = jnp.pad(a, ((0, 0), (0, n_mx - n0), (0, 0)))
    b = jnp.pad(b, ((0, 0), (0, n_mx - n1), (0, 0)))
    return jnp.concatenate([a, b], axis=0)

  src_p = _chunked(edge_index[0], 0)
  dst_p = _chunked(edge_index[1], n_pad - 1)

  zeros_d = jnp.zeros((slab, d_hid), jnp.float32)
  zeros_1d = jnp.zeros((n_pad,), jnp.float32)
  ones_ch = jnp.ones((_CH,), jnp.float32)
  batch2d = batch.reshape(1, n)
  b1r = b1.reshape(1, d_hid)
  b2r = b2.reshape(1, d_out)

  deg_raw = _degree_kernel(n_pad, n0, n1)(dst_p, ones_ch, zeros_1d)
  deg_p = deg_raw.reshape(_NC, n_pad, 1)

  agg = _edge_agg_kernel(n_pad, n0, n1, d_hid)

  hs1 = pl.pallas_call(
      _tc_first,
      out_shape=jax.ShapeDtypeStruct((n, d_hid), jnp.float32),
  )(x, W1, deg_p)

  agg1 = agg(hs1, src_p, dst_p, zeros_d)

  hs2 = pl.pallas_call(
      functools.partial(_tc_mid, n=n),
      out_shape=jax.ShapeDtypeStruct((n, d_out), jnp.float32),
  )(agg1, hs1, deg_p, b1r, W2)

  agg2 = agg(hs2, src_p, dst_p, zeros_d)

  out = pl.pallas_call(
      functools.partial(_tc_last, num_graphs=num_graphs, n=n),
      out_shape=jax.ShapeDtypeStruct((num_graphs, d_out), jnp.float32),
  )(agg2, hs2, deg_p, b2r, batch2d)

  return out


# speedup vs baseline: 1.2951x; 1.2951x over previous
import functools

import jax
import jax.numpy as jnp
from jax import lax
from jax.experimental import pallas as pl
from jax.experimental.pallas import tpu as pltpu
from jax.experimental.pallas import tpu_sc as plsc

_NC = 2
_NS = 16
_NW = _NC * _NS

_CH = 96
_NB = 3


def _edge_agg_kernel(n_pad, n0, n1, d):
  slab = n_pad // _NS
  mesh = plsc.VectorSubcoreMesh(core_axis_name="c", subcore_axis_name="s")

  nbi = 6
  nbd = 8

  @functools.partial(
      pl.kernel,
      out_type=jax.ShapeDtypeStruct((_NC, n_pad, d), jnp.float32),
      mesh=mesh,
      scratch_types=[
          pltpu.VMEM((nbi, _CH), jnp.int32),
          pltpu.VMEM((nbd, _CH), jnp.int32),
          pltpu.VMEM((_NB, _CH, d), jnp.float32),
          pltpu.VMEM_SHARED((n_pad, d), jnp.float32),
          pltpu.SemaphoreType.DMA((_NB,)),
          pltpu.SemaphoreType.DMA((_NB,)),
          pltpu.SemaphoreType.DMA((nbi,)),
          pltpu.SemaphoreType.DMA((nbd,)),
      ],
  )
  def k(hs_hbm, src_hbm, dst_hbm, zeros_hbm, out_hbm, idx_s, idx_d, rows,
        acc, sem_g, sem_s, sem_i, sem_j):
    c = lax.axis_index("c")
    s = lax.axis_index("s")
    wid = c * _NS + s
    nb = jnp.where(c == 0, n0, n1)
    pltpu.sync_copy(zeros_hbm, acc.at[pl.ds(s * slab, slab)])
    for bi in range(nbi):
      pltpu.async_copy(src_hbm.at[wid, bi], idx_s.at[bi], sem_i.at[bi])
    for bj in range(nbd):
      pltpu.async_copy(dst_hbm.at[wid, bj], idx_d.at[bj], sem_j.at[bj])
    plsc.subcore_barrier()

    for b in range(_NB - 1):
      pltpu.make_async_copy(src_hbm.at[wid, b], idx_s.at[b],
                            sem_i.at[b]).wait()
      pltpu.async_copy(hs_hbm.at[idx_s.at[b]], rows.at[b], sem_g.at[b])

    @pl.loop(0, nb)
    def _chunk(j):
      b = lax.rem(j, _NB)
      bi = lax.rem(j, nbi)
      bj = lax.rem(j, nbd)
      pltpu.make_async_copy(hs_hbm.at[idx_s.at[bi]], rows.at[b],
                            sem_g.at[b]).wait()
      prv = j - 1

      @pl.when(prv >= 0)
      def _wprev():
        bp = lax.rem(prv, _NB)
        bjp = lax.rem(prv, nbd)
        pltpu.make_async_copy(rows.at[bp], acc.at[idx_d.at[bjp]],
                              sem_s.at[bp]).wait()
        nxj = prv + nbd

        @pl.when(nxj < nb)
        def _pref_di():
          pltpu.async_copy(dst_hbm.at[wid, nxj], idx_d.at[bjp],
                           sem_j.at[bjp])

      pltpu.make_async_copy(dst_hbm.at[wid, j], idx_d.at[bj],
                            sem_j.at[bj]).wait()
      pltpu.async_copy(rows.at[b], acc.at[idx_d.at[bj]], sem_s.at[b],
                       add=True)
      nxi = j + nbi

      @pl.when(nxi < nb)
      def _pref_si():
        pltpu.async_copy(src_hbm.at[wid, nxi], idx_s.at[bi], sem_i.at[bi])

      nxg = j + _NB - 1

      @pl.when(nxg < nb)
      def _pref_rows():
        bg = lax.rem(nxg, _NB)
        big = lax.rem(nxg, nbi)
        pltpu.make_async_copy(src_hbm.at[wid, nxg], idx_s.at[big],
                              sem_i.at[big]).wait()
        pltpu.async_copy(hs_hbm.at[idx_s.at[big]], rows.at[bg],
                         sem_g.at[bg])

    lst = nb - 1
    pltpu.make_async_copy(rows.at[lax.rem(lst, _NB)],
                          acc.at[idx_d.at[lax.rem(lst, nbd)]],
                          sem_s.at[lax.rem(lst, _NB)]).wait()
    plsc.subcore_barrier()
    pltpu.sync_copy(acc.at[pl.ds(s * slab, slab)],
                    out_hbm.at[c, pl.ds(s * slab, slab)])

  return k


def _degree_kernel(n_pad, n0, n1):
  n_max = max(n0, n1)
  slab = n_pad // _NS
  nbd = 8
  mesh = plsc.VectorSubcoreMesh(core_axis_name="c", subcore_axis_name="s")

  @functools.partial(
      pl.kernel,
      out_type=jax.ShapeDtypeStruct((_NC, n_pad), jnp.float32),
      mesh=mesh,
      scratch_types=[
          pltpu.VMEM((n_max, _CH), jnp.int32),
          pltpu.VMEM((_CH,), jnp.float32),
          pltpu.VMEM_SHARED((n_pad,), jnp.float32),
          pltpu.SemaphoreType.DMA((nbd,)),
      ],
  )
  def k(dst_hbm, ones_hbm, zeros_hbm, out_hbm, idx_d, ones_v, acc, sem_s):
    c = lax.axis_index("c")
    s = lax.axis_index("s")
    wid = c * _NS + s
    nb = jnp.where(c == 0, n0, n1)
    pltpu.sync_copy(dst_hbm.at[wid], idx_d)
    pltpu.sync_copy(ones_hbm, ones_v)
    pltpu.sync_copy(zeros_hbm.at[pl.ds(0, slab)],
                    acc.at[pl.ds(s * slab, slab)])
    plsc.subcore_barrier()

    for b in range(nbd):
      pltpu.async_copy(ones_v, acc.at[idx_d.at[b]], sem_s.at[b], add=True)

    @pl.loop(0, nb)
    def _chunk(j):
      b = lax.rem(j, nbd)
      pltpu.make_async_copy(ones_v, acc.at[idx_d.at[j]], sem_s.at[b]).wait()
      nxt = j + nbd

      @pl.when(nxt < nb)
      def _nx():
        pltpu.async_copy(ones_v, acc.at[idx_d.at[nxt]], sem_s.at[b],
                         add=True)

    plsc.subcore_barrier()
    pltpu.sync_copy(acc.at[pl.ds(s * slab, slab)],
                    out_hbm.at[c, pl.ds(s * slab, slab)])

  return k


def _tc_first(x_ref, w_ref, degp_ref, out_ref):
  n = x_ref.shape[0]
  deg = degp_ref[0, :n, 0:1] + degp_ref[1, :n, 0:1] + 1.0
  g = lax.rsqrt(deg)
  h = jnp.dot(x_ref[...], w_ref[...], preferred_element_type=jnp.float32)
  out_ref[...] = h * g


def _tc_mid(aggp_ref, hs_ref, degp_ref, b_ref, w_ref, out_ref, *, n):
  deg = degp_ref[0, :n, 0:1] + degp_ref[1, :n, 0:1] + 1.0
  g = lax.rsqrt(deg)
  agg = aggp_ref[0, :n, :] + aggp_ref[1, :n, :] + hs_ref[...]
  h = jnp.maximum(agg * g + b_ref[...], 0.0)
  out_ref[...] = jnp.dot(h, w_ref[...],
                         preferred_element_type=jnp.float32) * g


def _tc_last(aggp_ref, hs_ref, degp_ref, b_ref, batch_ref, out_ref,
             *, num_graphs, n):
  deg = degp_ref[0, :n, 0:1] + degp_ref[1, :n, 0:1] + 1.0
  g = lax.rsqrt(deg)
  nodes = (aggp_ref[0, :n, :] + aggp_ref[1, :n, :] + hs_ref[...]) * g \
      + b_ref[...]
  gid = lax.broadcasted_iota(jnp.int32, (num_graphs, n), 0)
  mask = (gid == batch_ref[...]).astype(jnp.float32)
  sums = jnp.dot(mask, nodes, preferred_element_type=jnp.float32)
  cnt = jnp.sum(mask, axis=1, keepdims=True)
  out_ref[...] = sums / jnp.maximum(cnt, 1.0)


def kernel(x, edge_index, batch, W1, b1, W2, b2):
  n, d_in = x.shape
  d_hid = W1.shape[1]
  d_out = W2.shape[1]
  e = edge_index.shape[1]
  num_graphs = 64

  n_pad = ((n + 1023) // 1024) * 1024
  slab = n_pad // _NS

  grain = _NW * _CH
  e_pad = ((e + grain - 1) // grain) * grain
  tot_ch = e_pad // (_NS * _CH)
  n0 = (tot_ch * 4) // 5
  n1 = tot_ch - n0
  n_mx = max(n0, n1)

  def _chunked(flat, fill):
    flat = jnp.concatenate(
        [flat, jnp.full((e_pad - e,), fill, jnp.int32)])
    a = flat[: _NS * n0 * _CH].reshape(_NS, n0, _CH)
    b = flat[_NS * n0 * _CH:].reshape(_NS, n1, _CH)
    a = jnp.pad(a, ((0, 0), (0, n_mx - n0), (0, 0)))
    b = jnp.pad(b, ((0, 0), (0, n_mx - n1), (0, 0)))
    return jnp.concatenate([a, b], axis=0)

  src_p = _chunked(edge_index[0], 0)
  dst_p = _chunked(edge_index[1], n_pad - 1)

  zeros_d = jnp.zeros((slab, d_hid), jnp.float32)
  zeros_1d = jnp.zeros((n_pad,), jnp.float32)
  ones_ch = jnp.ones((_CH,), jnp.float32)
  batch2d = batch.reshape(1, n)
  b1r = b1.reshape(1, d_hid)
  b2r = b2.reshape(1, d_out)

  deg_raw = _degree_kernel(n_pad, n0, n1)(dst_p, ones_ch, zeros_1d)
  deg_p = deg_raw.reshape(_NC, n_pad, 1)

  agg = _edge_agg_kernel(n_pad, n0, n1, d_hid)

  hs1 = pl.pallas_call(
      _tc_first,
      out_shape=jax.ShapeDtypeStruct((n, d_hid), jnp.float32),
  )(x, W1, deg_p)

  agg1 = agg(hs1, src_p, dst_p, zeros_d)

  hs2 = pl.pallas_call(
      functools.partial(_tc_mid, n=n),
      out_shape=jax.ShapeDtypeStruct((n, d_out), jnp.float32),
  )(agg1, hs1, deg_p, b1r, W2)

  agg2 = agg(hs2, src_p, dst_p, zeros_d)

  out = pl.pallas_call(
      functools.partial(_tc_last, num_graphs=num_graphs, n=n),
      out_shape=jax.ShapeDtypeStruct((num_graphs, d_out), jnp.float32),
  )(agg2, hs2, deg_p, b2r, batch2d)

  return out
